# baseline (speedup 1.0000x reference)
import functools

import jax
import jax.numpy as jnp
from jax import lax
from jax.experimental import pallas as pl
from jax.experimental.pallas import tpu as pltpu
from jax.experimental.pallas import tpu_sc as plsc

KCODES = 8192
DIM = 256
BETA = 0.25

NB = 768
KB = 8192
NKB = KCODES // KB


def _argmin_body(with_sub, *refs):
    if with_sub:
        x_ref, q_ref, e_ref, o_ref, rmin, ridx, e2_s, ids_s = refs
    else:
        x_ref, e_ref, o_ref, rmin, ridx, e2_s, ids_s = refs
    i = pl.program_id(0)
    j = pl.program_id(1)
    if with_sub:
        x = x_ref[...] - q_ref[...]
    else:
        x = x_ref[...]
    f2 = jnp.sum(x * x, axis=1, keepdims=True)

    @pl.when(i == 0)
    def _():
        e = e_ref[...]
        e2_s[...] = jnp.sum(e * e, axis=1)[None, :]
        ids_s[...] = lax.broadcasted_iota(jnp.int32, (1, KB), 1).astype(
            jnp.float32)

    mm2 = lax.dot_general(x + x, e_ref[...], (((1,), (1,)), ((), ())),
                          preferred_element_type=jnp.float32)
    d = (f2 + e2_s[...]) - mm2
    bmin = jnp.min(d, axis=1, keepdims=True)
    wmin = jnp.min(jnp.where(d == bmin, ids_s[...], float(KCODES)),
                   axis=1, keepdims=True)
    barg = wmin.astype(jnp.int32) + j * KB

    @pl.when(j == 0)
    def _():
        rmin[...] = bmin
        ridx[...] = barg

    @pl.when(j > 0)
    def _():
        better = bmin < rmin[...]
        rmin[...] = jnp.where(better, bmin, rmin[...])
        ridx[...] = jnp.where(better, barg, ridx[...])

    @pl.when(j == NKB - 1)
    def _():
        o_ref[...] = ridx[...]


def _make_argmin(with_sub, n_rows):
    x_specs = [pl.BlockSpec((NB, DIM), lambda i, j: (i, 0))]
    if with_sub:
        x_specs.append(pl.BlockSpec((NB, DIM), lambda i, j: (i, 0)))
    return pl.pallas_call(
        functools.partial(_argmin_body, with_sub),
        grid=(n_rows // NB, NKB),
        in_specs=x_specs + [pl.BlockSpec((KB, DIM), lambda i, j: (j, 0))],
        out_specs=pl.BlockSpec((NB, 1), lambda i, j: (i, 0)),
        out_shape=jax.ShapeDtypeStruct((n_rows, 1), jnp.int32),
        scratch_shapes=[pltpu.VMEM((NB, 1), jnp.float32),
                        pltpu.VMEM((NB, 1), jnp.int32),
                        pltpu.VMEM((1, KB), jnp.float32),
                        pltpu.VMEM((1, KB), jnp.float32)],
        compiler_params=pltpu.CompilerParams(
            dimension_semantics=("arbitrary", "arbitrary")),
    )


def _make_sc_gather(n_rows):
    info = plsc.get_sparse_core_info()
    nw = info.num_cores * info.num_subcores
    b_per_w = n_rows // nw
    n_chunks = -(-b_per_w // 72)
    chunk = b_per_w // n_chunks
    assert chunk * n_chunks == b_per_w and chunk % 8 == 0
    mesh = plsc.VectorSubcoreMesh(core_axis_name="c", subcore_axis_name="s")

    @functools.partial(
        pl.kernel, mesh=mesh,
        out_type=jax.ShapeDtypeStruct((n_rows, DIM), jnp.float32),
        scratch_types=[pltpu.VMEM((chunk,), jnp.int32),
                       pltpu.VMEM((b_per_w, DIM), jnp.float32),
                       pltpu.SemaphoreType.DMA],
    )
    def gather(table_hbm, idx_hbm, out_hbm, idx_v, rows_v, sem):
        wid = lax.axis_index("s") * info.num_cores + lax.axis_index("c")
        base = wid * b_per_w
        for c in range(n_chunks):
            pltpu.sync_copy(idx_hbm.at[pl.ds(base + c * chunk, chunk)], idx_v)
            pltpu.async_copy(table_hbm.at[idx_v],
                             rows_v.at[pl.ds(c * chunk, chunk)], sem).wait()
        pltpu.sync_copy(rows_v, out_hbm.at[pl.ds(base, b_per_w)])

    return gather


def _combine_body(n_blocks, f_ref, a_ref, b_ref, qo_ref, lo_ref, acc):
    i = pl.program_id(0)
    q = a_ref[...] + b_ref[...]
    qo_ref[...] = q
    diff = q - f_ref[...]
    s = jnp.sum(diff * diff)

    @pl.when(i == 0)
    def _():
        acc[0] = 0.0

    acc[0] += s

    @pl.when(i == n_blocks - 1)
    def _():
        lo_ref[0, 0] = acc[0] * ((1.0 + BETA) / (n_blocks * NB * DIM))


def _make_combine(n_rows):
    blk = pl.BlockSpec((NB, DIM), lambda i: (i, 0))
    return pl.pallas_call(
        functools.partial(_combine_body, n_rows // NB),
        grid=(n_rows // NB,),
        in_specs=[blk, blk, blk],
        out_specs=[blk,
                   pl.BlockSpec(memory_space=pltpu.SMEM)],
        out_shape=[jax.ShapeDtypeStruct((n_rows, DIM), jnp.float32),
                   jax.ShapeDtypeStruct((1, 1), jnp.float32)],
        scratch_shapes=[pltpu.SMEM((1,), jnp.float32)],
        compiler_params=pltpu.CompilerParams(
            dimension_semantics=("arbitrary",)),
    )


def kernel(latents, emb1, emb2):
    shape = latents.shape
    flat = latents.reshape(-1, DIM)
    n_rows = flat.shape[0]

    argmin1 = _make_argmin(False, n_rows)
    argmin2 = _make_argmin(True, n_rows)
    sc_gather = _make_sc_gather(n_rows)
    combine = _make_combine(n_rows)

    idx1 = argmin1(flat, emb1)
    q1 = sc_gather(emb1, idx1.reshape(-1))
    idx2 = argmin2(flat, q1, emb2)
    q2 = sc_gather(emb2, idx2.reshape(-1))
    quant, vq_loss = combine(flat, q1, q2)

    return quant.reshape(shape), vq_loss.reshape(())


# speedup vs baseline: 1.1568x; 1.1568x over previous
import functools

import jax
import jax.numpy as jnp
from jax import lax
from jax.experimental import pallas as pl
from jax.experimental.pallas import tpu as pltpu
from jax.experimental.pallas import tpu_sc as plsc

KCODES = 8192
DIM = 256
BETA = 0.25

NB = 1152
KB = 8192
NKB = KCODES // KB


def _argmin_body(with_sub, *refs):
    if with_sub:
        x_ref, q_ref, e_ref, o_ref, rmin, ridx, e2_s, ids_s = refs
    else:
        x_ref, e_ref, o_ref, rmin, ridx, e2_s, ids_s = refs
    i = pl.program_id(0)
    j = pl.program_id(1)
    if with_sub:
        x = x_ref[...] - q_ref[...]
    else:
        x = x_ref[...]
    f2 = jnp.sum(x * x, axis=1, keepdims=True)

    @pl.when(i == 0)
    def _():
        e = e_ref[...]
        e2_s[...] = jnp.sum(e * e, axis=1)[None, :]
        ids_s[...] = lax.broadcasted_iota(jnp.int32, (1, KB), 1).astype(
            jnp.float32)

    mm = lax.dot_general(x, e_ref[...], (((1,), (1,)), ((), ())),
                         preferred_element_type=jnp.float32)
    d = (f2 + e2_s[...]) - 2.0 * mm
    bmin = jnp.min(d, axis=1, keepdims=True)
    wmin = jnp.min(jnp.where(d == bmin, ids_s[...], float(KCODES)),
                   axis=1, keepdims=True)
    barg = wmin.astype(jnp.int32) + j * KB

    @pl.when(j == 0)
    def _():
        rmin[...] = bmin
        ridx[...] = barg

    @pl.when(j > 0)
    def _():
        better = bmin < rmin[...]
        rmin[...] = jnp.where(better, bmin, rmin[...])
        ridx[...] = jnp.where(better, barg, ridx[...])

    @pl.when(j == NKB - 1)
    def _():
        o_ref[...] = ridx[...]


def _make_argmin(with_sub, n_rows):
    x_specs = [pl.BlockSpec((NB, DIM), lambda i, j: (i, 0))]
    if with_sub:
        x_specs.append(pl.BlockSpec((NB, DIM), lambda i, j: (i, 0)))
    return pl.pallas_call(
        functools.partial(_argmin_body, with_sub),
        grid=(n_rows // NB, NKB),
        in_specs=x_specs + [pl.BlockSpec((KB, DIM), lambda i, j: (j, 0))],
        out_specs=pl.BlockSpec((NB, 1), lambda i, j: (i, 0)),
        out_shape=jax.ShapeDtypeStruct((n_rows, 1), jnp.int32),
        scratch_shapes=[pltpu.VMEM((NB, 1), jnp.float32),
                        pltpu.VMEM((NB, 1), jnp.int32),
                        pltpu.VMEM((1, KB), jnp.float32),
                        pltpu.VMEM((1, KB), jnp.float32)],
        compiler_params=pltpu.CompilerParams(
            dimension_semantics=("arbitrary", "arbitrary")),
    )


def _make_sc_gather(n_rows):
    info = plsc.get_sparse_core_info()
    nw = info.num_cores * info.num_subcores
    b_per_w = n_rows // nw
    n_chunks = -(-b_per_w // 72)
    chunk = b_per_w // n_chunks
    assert chunk * n_chunks == b_per_w and chunk % 8 == 0
    mesh = plsc.VectorSubcoreMesh(core_axis_name="c", subcore_axis_name="s")

    @functools.partial(
        pl.kernel, mesh=mesh,
        out_type=jax.ShapeDtypeStruct((n_rows, DIM), jnp.float32),
        scratch_types=[pltpu.VMEM((chunk,), jnp.int32),
                       pltpu.VMEM((b_per_w, DIM), jnp.float32),
                       pltpu.SemaphoreType.DMA],
    )
    def gather(table_hbm, idx_hbm, out_hbm, idx_v, rows_v, sem):
        wid = lax.axis_index("s") * info.num_cores + lax.axis_index("c")
        base = wid * b_per_w
        for c in range(n_chunks):
            pltpu.sync_copy(idx_hbm.at[pl.ds(base + c * chunk, chunk)], idx_v)
            pltpu.async_copy(table_hbm.at[idx_v],
                             rows_v.at[pl.ds(c * chunk, chunk)], sem).wait()
        pltpu.sync_copy(rows_v, out_hbm.at[pl.ds(base, b_per_w)])

    return gather


def _combine_body(n_blocks, f_ref, a_ref, b_ref, qo_ref, lo_ref, acc):
    i = pl.program_id(0)
    q = a_ref[...] + b_ref[...]
    qo_ref[...] = q
    diff = q - f_ref[...]
    s = jnp.sum(diff * diff)

    @pl.when(i == 0)
    def _():
        acc[0] = 0.0

    acc[0] += s

    @pl.when(i == n_blocks - 1)
    def _():
        lo_ref[0, 0] = acc[0] * ((1.0 + BETA) / (n_blocks * NB * DIM))


def _make_combine(n_rows):
    blk = pl.BlockSpec((NB, DIM), lambda i: (i, 0))
    return pl.pallas_call(
        functools.partial(_combine_body, n_rows // NB),
        grid=(n_rows // NB,),
        in_specs=[blk, blk, blk],
        out_specs=[blk,
                   pl.BlockSpec(memory_space=pltpu.SMEM)],
        out_shape=[jax.ShapeDtypeStruct((n_rows, DIM), jnp.float32),
                   jax.ShapeDtypeStruct((1, 1), jnp.float32)],
        scratch_shapes=[pltpu.SMEM((1,), jnp.float32)],
        compiler_params=pltpu.CompilerParams(
            dimension_semantics=("arbitrary",)),
    )


def kernel(latents, emb1, emb2):
    shape = latents.shape
    flat = latents.reshape(-1, DIM)
    n_rows = flat.shape[0]

    argmin1 = _make_argmin(False, n_rows)
    argmin2 = _make_argmin(True, n_rows)
    sc_gather = _make_sc_gather(n_rows)
    combine = _make_combine(n_rows)

    idx1 = argmin1(flat, emb1)
    q1 = sc_gather(emb1, idx1.reshape(-1))
    idx2 = argmin2(flat, q1, emb2)
    q2 = sc_gather(emb2, idx2.reshape(-1))
    quant, vq_loss = combine(flat, q1, q2)

    return quant.reshape(shape), vq_loss.reshape(())
